# baseline (speedup 1.0000x reference)
import jax
import jax.numpy as jnp
from jax import lax
from jax.experimental import pallas as pl
from jax.experimental.pallas import tpu as pltpu
from jax.experimental.pallas import tpu_sc as plsc

BASE_VOCAB = 1000000
EXT_VOCAB = 1000
EMBED_DIM = 64
THRESHOLD = 1000000

NUM_CORES = 2
NUM_SUBCORES = 16
LANES = 16
NW = NUM_CORES * NUM_SUBCORES

CHUNK = 128
VPC = CHUNK // LANES
NBUF = 8
FIRE = NBUF - 2


def _body(tok_hbm, base_hbm, ext_hbm, out_hbm,
          tok_v, bidx_v, rows_v, patch_v, gsem, wsem, psem):
    n_tokens = tok_hbm.shape[0]
    per_w = n_tokens // NW
    n_chunks = per_w // CHUNK
    n_groups = n_chunks // NBUF

    wid = lax.axis_index("s") * NUM_CORES + lax.axis_index("c")
    base_off = wid * per_w

    pltpu.sync_copy(tok_hbm.at[pl.ds(base_off, per_w)], tok_v)

    lanes = lax.broadcasted_iota(jnp.int32, (LANES,), 0)

    def compute_bidx(c, b):
        tmax = jnp.zeros((LANES,), jnp.int32)
        for i in range(VPC):
            t = tok_v[pl.ds(c * CHUNK + i * LANES, LANES)]
            tmax = jnp.maximum(tmax, t)
            bidx_v[b, pl.ds(i * LANES, LANES)] = jnp.where(t >= THRESHOLD, 0, t)
        return jnp.max(tmax)

    def fire_gather(c, b):
        pltpu.async_copy(base_hbm.at[bidx_v.at[b]], rows_v.at[b], gsem.at[b])

    def wait_gather(b):
        pltpu.make_async_copy(base_hbm.at[bidx_v.at[b]], rows_v.at[b],
                              gsem.at[b]).wait()

    def fire_write(c, b):
        pltpu.async_copy(rows_v.at[b], out_hbm.at[pl.ds(base_off + c * CHUNK, CHUNK)],
                         wsem.at[b])

    def wait_write(b):
        pltpu.make_async_copy(rows_v.at[b], out_hbm.at[pl.ds(0, CHUNK)],
                              wsem.at[b]).wait()

    def patch_chunk(c, b):
        @pl.loop(0, VPC)
        def _vreg(i):
            t = tok_v[pl.ds(c * CHUNK + i * LANES, LANES)]
            m = t >= THRESHOLD

            @pl.when(jnp.max(t) >= THRESHOLD)
            def _patch():
                eidx = jnp.where(m, t - THRESHOLD, 0)
                pltpu.async_copy(ext_hbm.at[eidx], patch_v, psem).wait()
                row16 = i * LANES + lanes

                @pl.loop(0, EMBED_DIM)
                def _col(col):
                    col16 = jnp.full((LANES,), col, jnp.int32)
                    vals = plsc.load_gather(patch_v, [lanes, col16], mask=m)
                    plsc.store_scatter(rows_v.at[b], [row16, col16], vals, mask=m)

    for b in range(FIRE):
        compute_bidx(b, b)
        fire_gather(b, b)

    @pl.loop(0, n_groups)
    def _group(g):
        for b in range(NBUF):
            c = g * NBUF + b
            c_f = c + FIRE
            b_f = (b + FIRE) % NBUF

            @pl.when(c_f < n_chunks)
            def _fire():
                @pl.when(c_f >= NBUF)
                def _reuse():
                    wait_write(b_f)
                tmax_s = compute_bidx(c_f, b_f)
                del tmax_s
                fire_gather(c_f, b_f)

            wait_gather(b)

            tmax = jnp.int32(0)
            for i in range(VPC):
                t = tok_v[pl.ds(c * CHUNK + i * LANES, LANES)]
                tmax = jnp.maximum(tmax, jnp.max(t))

            @pl.when(tmax >= THRESHOLD)
            def _has_ext():
                patch_chunk(c, b)

            fire_write(c, b)

    for b in range(NBUF):
        wait_write(b)


@jax.jit
def _run(tok_flat, base_table, ext_table):
    mesh = plsc.VectorSubcoreMesh(
        core_axis_name="c", subcore_axis_name="s",
        num_cores=NUM_CORES, num_subcores=NUM_SUBCORES)
    per_w = tok_flat.shape[0] // NW
    f = pl.kernel(
        _body,
        out_type=jax.ShapeDtypeStruct((tok_flat.shape[0], EMBED_DIM), jnp.float32),
        mesh=mesh,
        scratch_types=[
            pltpu.VMEM((per_w,), jnp.int32),
            pltpu.VMEM((NBUF, CHUNK), jnp.int32),
            pltpu.VMEM((NBUF, CHUNK, EMBED_DIM), jnp.float32),
            pltpu.VMEM((LANES, EMBED_DIM), jnp.float32),
            pltpu.SemaphoreType.DMA((NBUF,)),
            pltpu.SemaphoreType.DMA((NBUF,)),
            pltpu.SemaphoreType.DMA,
        ],
        compiler_params=pltpu.CompilerParams(use_tc_tiling_on_sc=False,
                                             needs_layout_passes=False),
    )
    return f(tok_flat, base_table, ext_table)


def kernel(input_tokens, base_table, ext_table):
    b, s = input_tokens.shape
    out = _run(input_tokens.reshape(b * s), base_table, ext_table)
    return out.reshape(b, s, EMBED_DIM)


# speedup vs baseline: 1.0011x; 1.0011x over previous
import jax
import jax.numpy as jnp
from jax import lax
from jax.experimental import pallas as pl
from jax.experimental.pallas import tpu as pltpu
from jax.experimental.pallas import tpu_sc as plsc

BASE_VOCAB = 1000000
EXT_VOCAB = 1000
EMBED_DIM = 64
THRESHOLD = 1000000

NUM_CORES = 2
NUM_SUBCORES = 16
LANES = 16
NW = NUM_CORES * NUM_SUBCORES

CHUNK = 256
VPC = CHUNK // LANES
NBUF = 4
FIRE = NBUF - 2


def _body(tok_hbm, base_hbm, ext_hbm, out_hbm,
          tok_v, bidx_v, rows_v, patch_v, gsem, wsem, psem):
    n_tokens = tok_hbm.shape[0]
    per_w = n_tokens // NW
    n_chunks = per_w // CHUNK
    n_groups = n_chunks // NBUF

    wid = lax.axis_index("s") * NUM_CORES + lax.axis_index("c")
    base_off = wid * per_w

    pltpu.sync_copy(tok_hbm.at[pl.ds(base_off, per_w)], tok_v)

    lanes = lax.broadcasted_iota(jnp.int32, (LANES,), 0)

    def compute_bidx(c, b):
        tmax = jnp.zeros((LANES,), jnp.int32)
        for i in range(VPC):
            t = tok_v[pl.ds(c * CHUNK + i * LANES, LANES)]
            tmax = jnp.maximum(tmax, t)
            spread = base_off + c * CHUNK + i * LANES + lanes
            bidx_v[b, pl.ds(i * LANES, LANES)] = jnp.where(t >= THRESHOLD, spread, t)
        return jnp.max(tmax)

    def fire_gather(c, b):
        pltpu.async_copy(base_hbm.at[bidx_v.at[b]], rows_v.at[b], gsem.at[b])

    def wait_gather(b):
        pltpu.make_async_copy(base_hbm.at[bidx_v.at[b]], rows_v.at[b],
                              gsem.at[b]).wait()

    def fire_write(c, b):
        pltpu.async_copy(rows_v.at[b], out_hbm.at[pl.ds(base_off + c * CHUNK, CHUNK)],
                         wsem.at[b])

    def wait_write(b):
        pltpu.make_async_copy(rows_v.at[b], out_hbm.at[pl.ds(0, CHUNK)],
                              wsem.at[b]).wait()

    def patch_chunk(c, b):
        @pl.loop(0, VPC)
        def _vreg(i):
            t = tok_v[pl.ds(c * CHUNK + i * LANES, LANES)]
            m = t >= THRESHOLD

            @pl.when(jnp.max(t) >= THRESHOLD)
            def _patch():
                eidx = jnp.where(m, t - THRESHOLD, 0)
                pltpu.async_copy(ext_hbm.at[eidx], patch_v, psem).wait()
                row16 = i * LANES + lanes

                @pl.loop(0, EMBED_DIM)
                def _col(col):
                    col16 = jnp.full((LANES,), col, jnp.int32)
                    vals = plsc.load_gather(patch_v, [lanes, col16], mask=m)
                    plsc.store_scatter(rows_v.at[b], [row16, col16], vals, mask=m)

    for b in range(FIRE):
        compute_bidx(b, b)
        fire_gather(b, b)

    @pl.loop(0, n_groups)
    def _group(g):
        for b in range(NBUF):
            c = g * NBUF + b
            c_f = c + FIRE
            b_f = (b + FIRE) % NBUF

            @pl.when(c_f < n_chunks)
            def _fire():
                @pl.when(c_f >= NBUF)
                def _reuse():
                    wait_write(b_f)
                tmax_s = compute_bidx(c_f, b_f)
                del tmax_s
                fire_gather(c_f, b_f)

            wait_gather(b)

            tmax = jnp.int32(0)
            for i in range(VPC):
                t = tok_v[pl.ds(c * CHUNK + i * LANES, LANES)]
                tmax = jnp.maximum(tmax, jnp.max(t))

            @pl.when(tmax >= THRESHOLD)
            def _has_ext():
                patch_chunk(c, b)

            fire_write(c, b)

    for b in range(NBUF):
        wait_write(b)


@jax.jit
def _run(tok_flat, base_table, ext_table):
    mesh = plsc.VectorSubcoreMesh(
        core_axis_name="c", subcore_axis_name="s",
        num_cores=NUM_CORES, num_subcores=NUM_SUBCORES)
    per_w = tok_flat.shape[0] // NW
    f = pl.kernel(
        _body,
        out_type=jax.ShapeDtypeStruct((tok_flat.shape[0], EMBED_DIM), jnp.float32),
        mesh=mesh,
        scratch_types=[
            pltpu.VMEM((per_w,), jnp.int32),
            pltpu.VMEM((NBUF, CHUNK), jnp.int32),
            pltpu.VMEM((NBUF, CHUNK, EMBED_DIM), jnp.float32),
            pltpu.VMEM((LANES, EMBED_DIM), jnp.float32),
            pltpu.SemaphoreType.DMA((NBUF,)),
            pltpu.SemaphoreType.DMA((NBUF,)),
            pltpu.SemaphoreType.DMA,
        ],
        compiler_params=pltpu.CompilerParams(use_tc_tiling_on_sc=False,
                                             needs_layout_passes=False),
    )
    return f(tok_flat, base_table, ext_table)


def kernel(input_tokens, base_table, ext_table):
    b, s = input_tokens.shape
    out = _run(input_tokens.reshape(b * s), base_table, ext_table)
    return out.reshape(b, s, EMBED_DIM)
